# Initial kernel scaffold; baseline (speedup 1.0000x reference)
import jax
import jax.numpy as jnp
from jax.experimental import pallas as pl


def kernel(nodes, edges, senders, receivers, control, W_J, W_R, W_g, params):
    raise NotImplementedError("write your pallas kernel here")


# speedup vs baseline: 1.0208x; 1.0208x over previous
import functools
import jax
import jax.numpy as jnp
from jax import lax
from jax.experimental import pallas as pl
from jax.experimental.pallas import tpu as pltpu
from jax.experimental.pallas import tpu_sc as plsc

_E = 4096
_N = 2048
_L = 128
_DT = 0.01
_BLK = 512
_NB = _E // _BLK
_NC = 2
_NS = 16
_NW = _NC * _NS
_RPW = _E // _NW
_F32 = jnp.float32
_HI = lax.Precision.HIGHEST
_IT = False


def _ln_fwd(y, g, b):
    mu = jnp.mean(y, axis=-1, keepdims=True)
    yc = y - mu
    var = jnp.mean(yc * yc, axis=-1, keepdims=True)
    rstd = lax.rsqrt(var + 1e-6)
    xh = yc * rstd
    return xh * g + b, xh, rstd


def _ln_bwd(do, xh, rstd, g):
    dxh = do * g
    return rstd * (dxh - jnp.mean(dxh, axis=-1, keepdims=True)
                   - xh * jnp.mean(dxh * xh, axis=-1, keepdims=True))


def _mm(a, b):
    return jnp.dot(a, b, preferred_element_type=_F32, precision=_HI)


def _mmT(a, b):
    return lax.dot_general(a, b, (((1,), (1,)), ((), ())),
                           preferred_element_type=_F32, precision=_HI)


@functools.cache
def _build_gather2():
    @functools.partial(
        pl.kernel,
        out_type=[jax.ShapeDtypeStruct((_E, _L), _F32),
                  jax.ShapeDtypeStruct((_E, _L), _F32)],
        mesh=plsc.VectorSubcoreMesh(core_axis_name="c", subcore_axis_name="s"),
        scratch_types=[pltpu.VMEM((_RPW,), jnp.int32),
                       pltpu.VMEM((_RPW, _L), _F32),
                       pltpu.SemaphoreType.DMA],
    )
    def k(table, idxa, idxb, outa, outb, idx_v, rows_v, sem):
        wid = lax.axis_index("s") * _NC + lax.axis_index("c")
        base = wid * _RPW
        pltpu.sync_copy(idxa.at[pl.ds(base, _RPW)], idx_v)
        pltpu.async_copy(table.at[idx_v], rows_v, sem).wait()
        pltpu.sync_copy(rows_v, outa.at[pl.ds(base, _RPW)])
        pltpu.sync_copy(idxb.at[pl.ds(base, _RPW)], idx_v)
        pltpu.async_copy(table.at[idx_v], rows_v, sem).wait()
        pltpu.sync_copy(rows_v, outb.at[pl.ds(base, _RPW)])
    return k


@functools.cache
def _build_gather1():
    @functools.partial(
        pl.kernel,
        out_type=jax.ShapeDtypeStruct((_E, _L), _F32),
        mesh=plsc.VectorSubcoreMesh(core_axis_name="c", subcore_axis_name="s"),
        scratch_types=[pltpu.VMEM((_RPW,), jnp.int32),
                       pltpu.VMEM((_RPW, _L), _F32),
                       pltpu.SemaphoreType.DMA],
    )
    def k(table, idxa, outa, idx_v, rows_v, sem):
        wid = lax.axis_index("s") * _NC + lax.axis_index("c")
        base = wid * _RPW
        pltpu.sync_copy(idxa.at[pl.ds(base, _RPW)], idx_v)
        pltpu.async_copy(table.at[idx_v], rows_v, sem).wait()
        pltpu.sync_copy(rows_v, outa.at[pl.ds(base, _RPW)])
    return k


@functools.cache
def _build_scatter1():
    @functools.partial(
        pl.kernel,
        out_type=jax.ShapeDtypeStruct((2 * _N, _L), _F32),
        mesh=plsc.VectorSubcoreMesh(core_axis_name="c", subcore_axis_name="s"),
        scratch_types=[pltpu.VMEM_SHARED((_N, _L), _F32),
                       pltpu.VMEM((_RPW,), jnp.int32),
                       pltpu.VMEM((_RPW, _L), _F32)],
    )
    def k(zeros, vals, idx, out, shared, idx_v, rows_v):
        c = lax.axis_index("c")
        s = lax.axis_index("s")
        base = c * (_E // _NC) + s * _RPW

        @pl.when(s == 0)
        def _():
            pltpu.sync_copy(zeros, shared)

        plsc.subcore_barrier()
        pltpu.sync_copy(idx.at[pl.ds(base, _RPW)], idx_v)
        pltpu.sync_copy(vals.at[pl.ds(base, _RPW)], rows_v)
        pltpu.sync_copy(rows_v, shared.at[idx_v], add=True)
        plsc.subcore_barrier()
        rpo = _N // _NS
        pltpu.sync_copy(shared.at[pl.ds(s * rpo, rpo)],
                        out.at[pl.ds(c * _N + s * rpo, rpo)])
    return k


@functools.cache
def _build_scatter2():
    @functools.partial(
        pl.kernel,
        out_type=jax.ShapeDtypeStruct((2 * _N, _L), _F32),
        mesh=plsc.VectorSubcoreMesh(core_axis_name="c", subcore_axis_name="s"),
        scratch_types=[pltpu.VMEM_SHARED((_N, _L), _F32),
                       pltpu.VMEM((_RPW,), jnp.int32),
                       pltpu.VMEM((_RPW, _L), _F32)],
    )
    def k(zeros, valsa, idxa, valsb, idxb, out, shared, idx_v, rows_v):
        c = lax.axis_index("c")
        s = lax.axis_index("s")
        base = c * (_E // _NC) + s * _RPW

        @pl.when(s == 0)
        def _():
            pltpu.sync_copy(zeros, shared)

        plsc.subcore_barrier()
        pltpu.sync_copy(idxa.at[pl.ds(base, _RPW)], idx_v)
        pltpu.sync_copy(valsa.at[pl.ds(base, _RPW)], rows_v)
        pltpu.sync_copy(rows_v, shared.at[idx_v], add=True)
        pltpu.sync_copy(idxb.at[pl.ds(base, _RPW)], idx_v)
        pltpu.sync_copy(valsb.at[pl.ds(base, _RPW)], rows_v)
        pltpu.sync_copy(rows_v, shared.at[idx_v], add=True)
        plsc.subcore_barrier()
        rpo = _N // _NS
        pltpu.sync_copy(shared.at[pl.ds(s * rpo, rpo)],
                        out.at[pl.ds(c * _N + s * rpo, rpo)])
    return k


def _tcscat_kernel(ohr, vr, outr):
    v = vr[...]
    hi = v.astype(jnp.bfloat16)
    lo = (v - hi.astype(_F32)).astype(jnp.bfloat16)
    dn = (((0,), (0,)), ((), ()))
    outr[...] = (lax.dot_general(ohr[...], hi, dn, preferred_element_type=_F32)
                 + lax.dot_general(ohr[...], lo, dn, preferred_element_type=_F32))


def _tc_scatter(vals, idx):
    oh = jax.nn.one_hot(idx, _N, dtype=jnp.bfloat16)
    return pl.pallas_call(
        _tcscat_kernel,
        out_shape=jax.ShapeDtypeStruct((_N, _L), _F32),
        interpret=_IT,
    )(oh, vals)


def _sc_gather2(table, idxa, idxb):
    return _build_gather2()(table, idxa, idxb)


def _sc_gather1(table, idxa):
    return _build_gather1()(table, idxa)


def _sc_scatter1(zeros, vals, idx):
    return jnp.concatenate([_tc_scatter(vals, idx), zeros])


def _sc_scatter2(zeros, valsa, idxa, valsb, idxb):
    return jnp.concatenate([_tc_scatter(valsa, idxa), _tc_scatter(valsb, idxb)])


def _enc_kernel(xr, ndr, We1r, be1r, We2r, be2r, esr, ebr,
                Wn1r, bn1r, Wn2r, bn2r, nscr, nbir,
                el0r, nl0r, h1r, xher, rser):
    a1 = xr[...] * We1r[...] + be1r[...]
    h1 = jnp.maximum(a1, 0.0)
    a2 = _mm(h1, We2r[...]) + be2r[...]
    el0, xh, rs = _ln_fwd(a2, esr[...], ebr[...])
    el0r[...] = el0
    h1r[...] = h1
    xher[...] = xh
    rser[...] = rs
    nh = jnp.maximum(_mm(ndr[...], Wn1r[...]) + bn1r[...], 0.0)
    nl0, _, _ = _ln_fwd(_mm(nh, Wn2r[...]) + bn2r[...], nscr[...], nbir[...])
    nl0r[...] = nl0


def _proc_edge_kernel(elr, gSr, gRr, Wp1r, bp1r, Wp2r, bp2r, pscr, pbir,
                      elnr, q1r, xhr, rsr):
    el = elr[...]
    m = jnp.concatenate([el, gSr[...], gRr[...]], axis=-1)
    q1 = jnp.maximum(_mm(m, Wp1r[...]) + bp1r[...], 0.0)
    p2 = _mm(q1, Wp2r[...]) + bp2r[...]
    d, xh, rs = _ln_fwd(p2, pscr[...], pbir[...])
    elnr[...] = el + d
    q1r[...] = q1
    xhr[...] = xh
    rsr[...] = rs


def _proc_node_kernel(nlr, aggPr, Wq1r, bq1r, Wq2r, bq2r, qscr, qbir,
                      nlnr, s1r, xhr, rsr):
    nl = nlr[...]
    aggP = aggPr[...]
    agg = aggP[:_N] + aggP[_N:]
    c = jnp.concatenate([nl, agg], axis=-1)
    s1 = jnp.maximum(_mm(c, Wq1r[...]) + bq1r[...], 0.0)
    r2 = _mm(s1, Wq2r[...]) + bq2r[...]
    u, xh, rs = _ln_fwd(r2, qscr[...], qbir[...])
    nlnr[...] = nl + u
    s1r[...] = s1
    xhr[...] = xh
    rsr[...] = rs


def _dec_bwd_kernel(el2r, q1r, xhpr, rspr, Wd1r, bd1r, Wd2r, bd2r, Wd3r,
                    Wp1r, Wp2r, pscr, delr, dmsr, dmrr):
    el2 = el2r[...]
    z1 = _mm(el2, Wd1r[...]) + bd1r[...]
    z2 = _mm(jnp.maximum(z1, 0.0), Wd2r[...]) + bd2r[...]
    wd3row = Wd3r[...].reshape(1, _L)
    d_z2r = jnp.where(z2 > 0, wd3row, 0.0)
    d_z1 = _mmT(d_z2r, Wd2r[...]) * (z1 > 0).astype(_F32)
    d_el2 = _mmT(d_z1, Wd1r[...])
    d_p2 = _ln_bwd(d_el2, xhpr[...], rspr[...], pscr[...])
    d_m = _mmT(_mmT(d_p2, Wp2r[...]) * (q1r[...] > 0).astype(_F32), Wp1r[...])
    delr[...] = d_el2 + d_m[:, :_L]
    dmsr[...] = d_m[:, _L:2 * _L]
    dmrr[...] = d_m[:, 2 * _L:]


def _node_bwd_kernel(dnlPr, s1r, xhr, rsr, Wq1r, Wq2r, qscr, daggr):
    dnlP = dnlPr[...]
    d_nl = dnlP[:_N] + dnlP[_N:]
    d_r2 = _ln_bwd(d_nl, xhr[...], rsr[...], qscr[...])
    d_c = _mmT(_mmT(d_r2, Wq2r[...]) * (s1r[...] > 0).astype(_F32), Wq1r[...])
    daggr[...] = d_c[:, _L:]


def _final_bwd_kernel(delr, gdaggr, q1r, xhpr, rspr, h1r, xher, rser,
                      Wp1r, Wp2r, pscr, We1r, We2r, esr, dhr):
    d_el_tot = delr[...] + gdaggr[...]
    d_p2 = _ln_bwd(d_el_tot, xhpr[...], rspr[...], pscr[...])
    d_m = _mmT(_mmT(d_p2, Wp2r[...]) * (q1r[...] > 0).astype(_F32), Wp1r[...])
    d_el0 = d_el_tot + d_m[:, :_L]
    d_a2 = _ln_bwd(d_el0, xher[...], rser[...], esr[...])
    d_a1 = _mmT(d_a2, We2r[...]) * (h1r[...] > 0).astype(_F32)
    dhr[...] = jnp.sum(d_a1 * We1r[...], axis=-1, keepdims=True)


def _pass1_kernel(wjr, wrr, wgr, vjr, vir, ctlr, accr, ur):
    i = pl.program_id(0)
    j = pl.program_id(1)

    @pl.when(jnp.logical_and(i == 0, j == 0))
    def _():
        accr[...] = jnp.zeros_like(accr)
        ur[...] = jnp.zeros_like(ur)

    rg = i * _BLK + lax.broadcasted_iota(jnp.int32, (_BLK, _BLK), 0)
    cg = j * _BLK + lax.broadcasted_iota(jnp.int32, (_BLK, _BLK), 1)
    wjm = jnp.where(rg <= cg, wjr[...], 0.0)
    wrm = jnp.where(rg >= cg, wrr[...], 0.0)
    vj = vjr[...]
    vi = vir[...]
    dnT = (((0,), (0,)), ((), ()))

    accr[pl.ds(i * _BLK, _BLK), :] += _mm(wjm, vj) + _mm(wgr[...], ctlr[...])
    t2 = lax.dot_general(wjm, vi, dnT, preferred_element_type=_F32, precision=_HI)
    uj = lax.dot_general(wrm, vi, dnT, preferred_element_type=_F32, precision=_HI)
    accr[pl.ds(j * _BLK, _BLK), :] += -t2
    ur[pl.ds(j * _BLK, _BLK), :] += uj


def _pass2_kernel(wrr, ujr, accr, xr, outr, rrr):
    i = pl.program_id(0)
    j = pl.program_id(1)

    @pl.when(jnp.logical_and(i == 0, j == 0))
    def _():
        rrr[...] = jnp.zeros_like(rrr)

    rg = i * _BLK + lax.broadcasted_iota(jnp.int32, (_BLK, _BLK), 0)
    cg = j * _BLK + lax.broadcasted_iota(jnp.int32, (_BLK, _BLK), 1)
    wrm = jnp.where(rg >= cg, wrr[...], 0.0)
    rrr[pl.ds(i * _BLK, _BLK), :] += _mm(wrm, ujr[...])

    @pl.when(jnp.logical_and(i == _NB - 1, j == _NB - 1))
    def _():
        outr[...] = xr[...] + _DT * (accr[...] - rrr[...])


def _tc_call(body, out_shape, n_in):
    return pl.pallas_call(body, out_shape=out_shape, interpret=_IT)


def kernel(nodes, edges, senders, receivers, control, W_J, W_R, W_g, params):
    x = edges[:, :1]
    snd = senders.astype(jnp.int32)
    rcv = receivers.astype(jnp.int32)
    zeros_n = jnp.zeros((_N, _L), _F32)

    pe, pn = params["enc_edge"], params["enc_node"]
    pp, pq, pd = params["proc_edge"], params["proc_node"], params["dec_edge"]
    We1, We2 = pe["W"]
    be1, be2 = pe["b"]
    es, eb = pe["ln_scale"], pe["ln_bias"]
    Wn1, Wn2 = pn["W"]
    bn1, bn2 = pn["b"]
    nsc, nbi = pn["ln_scale"], pn["ln_bias"]
    Wp1, Wp2 = pp["W"]
    bp1, bp2 = pp["b"]
    psc, pbi = pp["ln_scale"], pp["ln_bias"]
    Wq1, Wq2 = pq["W"]
    bq1, bq2 = pq["b"]
    qsc, qbi = pq["ln_scale"], pq["ln_bias"]
    Wd1, Wd2, Wd3 = pd["W"]
    bd1, bd2 = pd["b"][0], pd["b"][1]

    f_el = jax.ShapeDtypeStruct((_E, _L), _F32)
    f_nl = jax.ShapeDtypeStruct((_N, _L), _F32)
    f_e1 = jax.ShapeDtypeStruct((_E, 1), _F32)

    el0, nl0, h1, xh_e, rs_e = pl.pallas_call(
        _enc_kernel,
        out_shape=[f_el, f_nl, f_el, f_el, f_e1],
        interpret=_IT,
    )(x, nodes, We1, be1, We2, be2, es, eb, Wn1, bn1, Wn2, bn2, nsc, nbi)

    g0S, g0R = _sc_gather2(nl0, snd, rcv)

    el1, q1_0, xh_p0, rs_p0 = pl.pallas_call(
        _proc_edge_kernel,
        out_shape=[f_el, f_el, f_el, f_e1],
        interpret=_IT,
    )(el0, g0S, g0R, Wp1, bp1, Wp2, bp2, psc, pbi)

    aggP = _sc_scatter1(zeros_n, el1, rcv)

    nl1, s1_0, xh_r0, rs_r0 = pl.pallas_call(
        _proc_node_kernel,
        out_shape=[f_nl, f_nl, f_nl, jax.ShapeDtypeStruct((_N, 1), _F32)],
        interpret=_IT,
    )(nl0, aggP, Wq1, bq1, Wq2, bq2, qsc, qbi)

    g1S, g1R = _sc_gather2(nl1, snd, rcv)

    el2, q1_1, xh_p1, rs_p1 = pl.pallas_call(
        _proc_edge_kernel,
        out_shape=[f_el, f_el, f_el, f_e1],
        interpret=_IT,
    )(el1, g1S, g1R, Wp1, bp1, Wp2, bp2, psc, pbi)

    d_el1, d_mS, d_mR = pl.pallas_call(
        _dec_bwd_kernel,
        out_shape=[f_el, f_el, f_el],
        interpret=_IT,
    )(el2, q1_1, xh_p1, rs_p1, Wd1, bd1, Wd2, bd2, Wd3, Wp1, Wp2, psc)

    dnlP = _sc_scatter2(zeros_n, d_mS, snd, d_mR, rcv)

    d_agg = pl.pallas_call(
        _node_bwd_kernel,
        out_shape=f_nl,
        interpret=_IT,
    )(dnlP, s1_0, xh_r0, rs_r0, Wq1, Wq2, qsc)

    gDagg = _sc_gather1(d_agg, rcv)

    dH = pl.pallas_call(
        _final_bwd_kernel,
        out_shape=f_e1,
        interpret=_IT,
    )(d_el1, gDagg, q1_0, xh_p0, rs_p0, h1, xh_e, rs_e,
      Wp1, Wp2, psc, We1, We2, es)

    ctl = control[:, None]
    bspec_mat = pl.BlockSpec((_BLK, _BLK), lambda i, j: (i, j))
    bspec_vj = pl.BlockSpec((_BLK, 1), lambda i, j: (j, 0))
    bspec_vi = pl.BlockSpec((_BLK, 1), lambda i, j: (i, 0))
    bspec_full = pl.BlockSpec((_E, 1), lambda i, j: (0, 0))

    acc, u = pl.pallas_call(
        _pass1_kernel,
        grid=(_NB, _NB),
        in_specs=[bspec_mat, bspec_mat, bspec_mat, bspec_vj, bspec_vi, bspec_vj],
        out_specs=[bspec_full, bspec_full],
        out_shape=[f_e1, f_e1],
        interpret=_IT,
    )(W_J, W_R, W_g, dH, dH, ctl)

    nxt = pl.pallas_call(
        _pass2_kernel,
        grid=(_NB, _NB),
        in_specs=[bspec_mat, bspec_vj, bspec_full, bspec_full],
        out_specs=bspec_full,
        out_shape=f_e1,
        scratch_shapes=[pltpu.VMEM((_E, 1), _F32)],
        interpret=_IT,
    )(W_R, u, acc, x)

    return nxt[:, 0]


# speedup vs baseline: 1.2747x; 1.2487x over previous
import functools
import jax
import jax.numpy as jnp
from jax import lax
from jax.experimental import pallas as pl
from jax.experimental.pallas import tpu as pltpu
from jax.experimental.pallas import tpu_sc as plsc

_E = 4096
_N = 2048
_L = 128
_DT = 0.01
_BLK = 512
_NB = _E // _BLK
_NC = 2
_NS = 16
_NW = _NC * _NS
_RPW = _E // _NW
_F32 = jnp.float32
_HI = lax.Precision.HIGHEST
_IT = False


def _ln_fwd(y, g, b):
    mu = jnp.mean(y, axis=-1, keepdims=True)
    yc = y - mu
    var = jnp.mean(yc * yc, axis=-1, keepdims=True)
    rstd = lax.rsqrt(var + 1e-6)
    xh = yc * rstd
    return xh * g + b, xh, rstd


def _ln_bwd(do, xh, rstd, g):
    dxh = do * g
    return rstd * (dxh - jnp.mean(dxh, axis=-1, keepdims=True)
                   - xh * jnp.mean(dxh * xh, axis=-1, keepdims=True))


def _mm(a, b):
    return jnp.dot(a, b, preferred_element_type=_F32, precision=_HI)


def _mmT(a, b):
    return lax.dot_general(a, b, (((1,), (1,)), ((), ())),
                           preferred_element_type=_F32, precision=_HI)


def _seg_sum(idx, vals):
    oht = (idx[None, :] == lax.broadcasted_iota(jnp.int32, (_N, _E), 0)
           ).astype(jnp.bfloat16)
    hi = vals.astype(jnp.bfloat16)
    lo = (vals - hi.astype(_F32)).astype(jnp.bfloat16)
    return (jnp.dot(oht, hi, preferred_element_type=_F32)
            + jnp.dot(oht, lo, preferred_element_type=_F32))


@functools.cache
def _build_gather2():
    @functools.partial(
        pl.kernel,
        out_type=[jax.ShapeDtypeStruct((_E, _L), _F32),
                  jax.ShapeDtypeStruct((_E, _L), _F32)],
        mesh=plsc.VectorSubcoreMesh(core_axis_name="c", subcore_axis_name="s"),
        scratch_types=[pltpu.VMEM((_RPW,), jnp.int32),
                       pltpu.VMEM((_RPW, _L), _F32),
                       pltpu.SemaphoreType.DMA],
    )
    def k(table, idxa, idxb, outa, outb, idx_v, rows_v, sem):
        wid = lax.axis_index("s") * _NC + lax.axis_index("c")
        base = wid * _RPW
        pltpu.sync_copy(idxa.at[pl.ds(base, _RPW)], idx_v)
        pltpu.async_copy(table.at[idx_v], rows_v, sem).wait()
        pltpu.sync_copy(rows_v, outa.at[pl.ds(base, _RPW)])
        pltpu.sync_copy(idxb.at[pl.ds(base, _RPW)], idx_v)
        pltpu.async_copy(table.at[idx_v], rows_v, sem).wait()
        pltpu.sync_copy(rows_v, outb.at[pl.ds(base, _RPW)])
    return k


@functools.cache
def _build_gather1():
    @functools.partial(
        pl.kernel,
        out_type=jax.ShapeDtypeStruct((_E, _L), _F32),
        mesh=plsc.VectorSubcoreMesh(core_axis_name="c", subcore_axis_name="s"),
        scratch_types=[pltpu.VMEM((_RPW,), jnp.int32),
                       pltpu.VMEM((_RPW, _L), _F32),
                       pltpu.SemaphoreType.DMA],
    )
    def k(table, idxa, outa, idx_v, rows_v, sem):
        wid = lax.axis_index("s") * _NC + lax.axis_index("c")
        base = wid * _RPW
        pltpu.sync_copy(idxa.at[pl.ds(base, _RPW)], idx_v)
        pltpu.async_copy(table.at[idx_v], rows_v, sem).wait()
        pltpu.sync_copy(rows_v, outa.at[pl.ds(base, _RPW)])
    return k


def _sc_gather2(table, idxa, idxb):
    return _build_gather2()(table, idxa, idxb)


def _sc_gather1(table, idxa):
    return _build_gather1()(table, idxa)


def _enc_kernel(xr, ndr, We1r, be1r, We2r, be2r, esr, ebr,
                Wn1r, bn1r, Wn2r, bn2r, nscr, nbir,
                el0r, nl0r, h1r, xher, rser):
    a1 = xr[...] * We1r[...] + be1r[...]
    h1 = jnp.maximum(a1, 0.0)
    a2 = _mm(h1, We2r[...]) + be2r[...]
    el0, xh, rs = _ln_fwd(a2, esr[...], ebr[...])
    el0r[...] = el0
    h1r[...] = h1
    xher[...] = xh
    rser[...] = rs
    nh = jnp.maximum(_mm(ndr[...], Wn1r[...]) + bn1r[...], 0.0)
    nl0, _, _ = _ln_fwd(_mm(nh, Wn2r[...]) + bn2r[...], nscr[...], nbir[...])
    nl0r[...] = nl0


def _proc_edge_kernel(elr, gSr, gRr, Wp1r, bp1r, Wp2r, bp2r, pscr, pbir,
                      elnr, q1r, xhr, rsr):
    el = elr[...]
    m = jnp.concatenate([el, gSr[...], gRr[...]], axis=-1)
    q1 = jnp.maximum(_mm(m, Wp1r[...]) + bp1r[...], 0.0)
    p2 = _mm(q1, Wp2r[...]) + bp2r[...]
    d, xh, rs = _ln_fwd(p2, pscr[...], pbir[...])
    elnr[...] = el + d
    q1r[...] = q1
    xhr[...] = xh
    rsr[...] = rs


def _proc_node_kernel(nlr, el1r, rcvr, Wq1r, bq1r, Wq2r, bq2r, qscr, qbir,
                      nlnr, s1r, xhr, rsr):
    nl = nlr[...]
    agg = _seg_sum(rcvr[...], el1r[...])
    c = jnp.concatenate([nl, agg], axis=-1)
    s1 = jnp.maximum(_mm(c, Wq1r[...]) + bq1r[...], 0.0)
    r2 = _mm(s1, Wq2r[...]) + bq2r[...]
    u, xh, rs = _ln_fwd(r2, qscr[...], qbir[...])
    nlnr[...] = nl + u
    s1r[...] = s1
    xhr[...] = xh
    rsr[...] = rs


def _edge_dec_bwd_kernel(elr, gSr, gRr, Wp1r, bp1r, Wp2r, bp2r, pscr, pbir,
                         Wd1r, bd1r, Wd2r, bd2r, Wd3r,
                         delr, dmsr, dmrr):
    el = elr[...]
    Wp1 = Wp1r[...]
    Wp2 = Wp2r[...]
    psc = pscr[...]
    m = jnp.concatenate([el, gSr[...], gRr[...]], axis=-1)
    q1 = jnp.maximum(_mm(m, Wp1) + bp1r[...], 0.0)
    p2 = _mm(q1, Wp2) + bp2r[...]
    d, xh_p, rs_p = _ln_fwd(p2, psc, pbir[...])
    el2 = el + d
    z1 = _mm(el2, Wd1r[...]) + bd1r[...]
    z2 = _mm(jnp.maximum(z1, 0.0), Wd2r[...]) + bd2r[...]
    wd3row = Wd3r[...].reshape(1, _L)
    d_z2r = jnp.where(z2 > 0, wd3row, 0.0)
    d_z1 = _mmT(d_z2r, Wd2r[...]) * (z1 > 0).astype(_F32)
    d_el2 = _mmT(d_z1, Wd1r[...])
    d_p2 = _ln_bwd(d_el2, xh_p, rs_p, psc)
    d_m = _mmT(_mmT(d_p2, Wp2) * (q1 > 0).astype(_F32), Wp1)
    delr[...] = d_el2 + d_m[:, :_L]
    dmsr[...] = d_m[:, _L:2 * _L]
    dmrr[...] = d_m[:, 2 * _L:]


def _node_bwd_kernel(dmsr, dmrr, sndr, rcvr, s1r, xhr, rsr,
                     Wq1r, Wq2r, qscr, daggr):
    d_nl = _seg_sum(sndr[...], dmsr[...]) + _seg_sum(rcvr[...], dmrr[...])
    d_r2 = _ln_bwd(d_nl, xhr[...], rsr[...], qscr[...])
    d_c = _mmT(_mmT(d_r2, Wq2r[...]) * (s1r[...] > 0).astype(_F32), Wq1r[...])
    daggr[...] = d_c[:, _L:]


def _final_bwd_kernel(delr, gdaggr, q1r, xhpr, rspr, h1r, xher, rser,
                      Wp1r, Wp2r, pscr, We1r, We2r, esr, dhr):
    d_el_tot = delr[...] + gdaggr[...]
    d_p2 = _ln_bwd(d_el_tot, xhpr[...], rspr[...], pscr[...])
    d_m = _mmT(_mmT(d_p2, Wp2r[...]) * (q1r[...] > 0).astype(_F32), Wp1r[...])
    d_el0 = d_el_tot + d_m[:, :_L]
    d_a2 = _ln_bwd(d_el0, xher[...], rser[...], esr[...])
    d_a1 = _mmT(d_a2, We2r[...]) * (h1r[...] > 0).astype(_F32)
    dhr[...] = jnp.sum(d_a1 * We1r[...], axis=-1, keepdims=True)


def _mv1_kernel(wjr, wrr, wgr, vjr, vir, ctlr, accr, ur):
    i = pl.program_id(0)
    j = pl.program_id(1)

    @pl.when(jnp.logical_and(i == 0, j == 0))
    def _():
        accr[...] = jnp.zeros_like(accr)
        ur[...] = jnp.zeros_like(ur)

    accr[pl.ds(i * _BLK, _BLK), :] += _mm(wgr[...], ctlr[...])

    @pl.when(i <= j)
    def _():
        rg = i * _BLK + lax.broadcasted_iota(jnp.int32, (_BLK, _BLK), 0)
        cg = j * _BLK + lax.broadcasted_iota(jnp.int32, (_BLK, _BLK), 1)
        wjm = jnp.where(rg <= cg, wjr[...], 0.0)
        rgr = j * _BLK + lax.broadcasted_iota(jnp.int32, (_BLK, _BLK), 0)
        cgr = i * _BLK + lax.broadcasted_iota(jnp.int32, (_BLK, _BLK), 1)
        wrm = jnp.where(rgr >= cgr, wrr[...], 0.0)
        vj = vjr[...]
        vi = vir[...]
        dnT = (((0,), (0,)), ((), ()))
        accr[pl.ds(i * _BLK, _BLK), :] += _mm(wjm, vj)
        accr[pl.ds(j * _BLK, _BLK), :] += -lax.dot_general(
            wjm, vi, dnT, preferred_element_type=_F32, precision=_HI)
        ur[pl.ds(i * _BLK, _BLK), :] += lax.dot_general(
            wrm, vj, dnT, preferred_element_type=_F32, precision=_HI)


def _mv2_kernel(wrr, ujr, accr, xr, outr, rrr):
    i = pl.program_id(0)
    j = pl.program_id(1)

    @pl.when(jnp.logical_and(i == 0, j == 0))
    def _():
        rrr[...] = jnp.zeros_like(rrr)

    @pl.when(j <= i)
    def _():
        rg = i * _BLK + lax.broadcasted_iota(jnp.int32, (_BLK, _BLK), 0)
        cg = j * _BLK + lax.broadcasted_iota(jnp.int32, (_BLK, _BLK), 1)
        wrm = jnp.where(rg >= cg, wrr[...], 0.0)
        rrr[pl.ds(i * _BLK, _BLK), :] += _mm(wrm, ujr[...])

    @pl.when(jnp.logical_and(i == _NB - 1, j == _NB - 1))
    def _():
        outr[...] = xr[...] + _DT * (accr[...] - rrr[...])


def kernel(nodes, edges, senders, receivers, control, W_J, W_R, W_g, params):
    x = edges[:, :1]
    snd = senders.astype(jnp.int32)
    rcv = receivers.astype(jnp.int32)

    pe, pn = params["enc_edge"], params["enc_node"]
    pp, pq, pd = params["proc_edge"], params["proc_node"], params["dec_edge"]
    We1, We2 = pe["W"]
    be1, be2 = pe["b"]
    es, eb = pe["ln_scale"], pe["ln_bias"]
    Wn1, Wn2 = pn["W"]
    bn1, bn2 = pn["b"]
    nsc, nbi = pn["ln_scale"], pn["ln_bias"]
    Wp1, Wp2 = pp["W"]
    bp1, bp2 = pp["b"]
    psc, pbi = pp["ln_scale"], pp["ln_bias"]
    Wq1, Wq2 = pq["W"]
    bq1, bq2 = pq["b"]
    qsc, qbi = pq["ln_scale"], pq["ln_bias"]
    Wd1, Wd2, Wd3 = pd["W"]
    bd1, bd2 = pd["b"][0], pd["b"][1]

    f_el = jax.ShapeDtypeStruct((_E, _L), _F32)
    f_nl = jax.ShapeDtypeStruct((_N, _L), _F32)
    f_e1 = jax.ShapeDtypeStruct((_E, 1), _F32)

    el0, nl0, h1, xh_e, rs_e = pl.pallas_call(
        _enc_kernel,
        out_shape=[f_el, f_nl, f_el, f_el, f_e1],
        interpret=_IT,
    )(x, nodes, We1, be1, We2, be2, es, eb, Wn1, bn1, Wn2, bn2, nsc, nbi)

    g0S, g0R = _sc_gather2(nl0, snd, rcv)

    el1, q1_0, xh_p0, rs_p0 = pl.pallas_call(
        _proc_edge_kernel,
        out_shape=[f_el, f_el, f_el, f_e1],
        interpret=_IT,
    )(el0, g0S, g0R, Wp1, bp1, Wp2, bp2, psc, pbi)

    nl1, s1_0, xh_r0, rs_r0 = pl.pallas_call(
        _proc_node_kernel,
        out_shape=[f_nl, f_nl, f_nl, jax.ShapeDtypeStruct((_N, 1), _F32)],
        interpret=_IT,
    )(nl0, el1, rcv, Wq1, bq1, Wq2, bq2, qsc, qbi)

    g1S, g1R = _sc_gather2(nl1, snd, rcv)

    d_el1, d_mS, d_mR = pl.pallas_call(
        _edge_dec_bwd_kernel,
        out_shape=[f_el, f_el, f_el],
        interpret=_IT,
    )(el1, g1S, g1R, Wp1, bp1, Wp2, bp2, psc, pbi,
      Wd1, bd1, Wd2, bd2, Wd3)

    d_agg = pl.pallas_call(
        _node_bwd_kernel,
        out_shape=f_nl,
        interpret=_IT,
    )(d_mS, d_mR, snd, rcv, s1_0, xh_r0, rs_r0, Wq1, Wq2, qsc)

    gDagg = _sc_gather1(d_agg, rcv)

    dH = pl.pallas_call(
        _final_bwd_kernel,
        out_shape=f_e1,
        interpret=_IT,
    )(d_el1, gDagg, q1_0, xh_p0, rs_p0, h1, xh_e, rs_e,
      Wp1, Wp2, psc, We1, We2, es)

    ctl = control[:, None]
    spec_wj = pl.BlockSpec(
        (_BLK, _BLK),
        lambda i, j: (jnp.where(i <= j, i, i - 1), jnp.where(i <= j, j, _NB - 1)))
    spec_wr1 = pl.BlockSpec(
        (_BLK, _BLK),
        lambda i, j: (jnp.where(i <= j, j, _NB - 1), jnp.where(i <= j, i, i - 1)))
    spec_wg = pl.BlockSpec((_BLK, _BLK), lambda i, j: (i, j))
    spec_wr2 = pl.BlockSpec((_BLK, _BLK), lambda i, j: (i, jnp.minimum(j, i)))
    spec_vj = pl.BlockSpec((_BLK, 1), lambda i, j: (j, 0))
    spec_vi = pl.BlockSpec((_BLK, 1), lambda i, j: (i, 0))
    spec_full = pl.BlockSpec((_E, 1), lambda i, j: (0, 0))

    acc, u = pl.pallas_call(
        _mv1_kernel,
        grid=(_NB, _NB),
        in_specs=[spec_wj, spec_wr1, spec_wg, spec_vj, spec_vi, spec_vj],
        out_specs=[spec_full, spec_full],
        out_shape=[f_e1, f_e1],
        interpret=_IT,
    )(W_J, W_R, W_g, dH, dH, ctl)

    nxt = pl.pallas_call(
        _mv2_kernel,
        grid=(_NB, _NB),
        in_specs=[spec_wr2, spec_vj, spec_full, spec_full],
        out_specs=spec_full,
        out_shape=f_e1,
        scratch_shapes=[pltpu.VMEM((_E, 1), _F32)],
        interpret=_IT,
    )(W_R, u, acc, x)

    return nxt[:, 0]


# speedup vs baseline: 1.6163x; 1.2679x over previous
import functools
import jax
import jax.numpy as jnp
from jax import lax
from jax.experimental import pallas as pl
from jax.experimental.pallas import tpu as pltpu
from jax.experimental.pallas import tpu_sc as plsc

_E = 4096
_N = 2048
_L = 128
_DT = 0.01
_BLK = 1024
_NB = _E // _BLK
_T1 = _NB * (_NB + 1) // 2
_NC = 2
_NS = 16
_NW = _NC * _NS
_RPW = _E // _NW
_F32 = jnp.float32
_HI = lax.Precision.HIGHEST
_IT = False


def _ln_fwd(y, g, b):
    mu = jnp.mean(y, axis=-1, keepdims=True)
    yc = y - mu
    var = jnp.mean(yc * yc, axis=-1, keepdims=True)
    s = jnp.sqrt(var + 1e-6)
    xh = yc / s
    return xh * g + b, xh, 1.0 / s


def _ln_bwd(do, xh, rstd, g):
    dxh = do * g
    return rstd * (dxh - jnp.mean(dxh, axis=-1, keepdims=True)
                   - xh * jnp.mean(dxh * xh, axis=-1, keepdims=True))


def _mm(a, b):
    return jnp.dot(a, b, preferred_element_type=_F32, precision=_HI)


def _mmT(a, b):
    return lax.dot_general(a, b, (((1,), (1,)), ((), ())),
                           preferred_element_type=_F32, precision=_HI)


def _b16(a):
    return a.astype(jnp.bfloat16)


def _mmb(a, b):
    return jnp.dot(_b16(a), _b16(b), preferred_element_type=_F32)


def _mmbT(a, b):
    return lax.dot_general(_b16(a), _b16(b), (((1,), (1,)), ((), ())),
                           preferred_element_type=_F32)


def _seg_sum(idx, vals, exact=False):
    oht = (idx[None, :] == lax.broadcasted_iota(jnp.int32, (_N, _E), 0)
           ).astype(jnp.bfloat16)
    hi = vals.astype(jnp.bfloat16)
    r1 = vals - hi.astype(_F32)
    lo = r1.astype(jnp.bfloat16)
    out = (jnp.dot(oht, hi, preferred_element_type=_F32)
           + jnp.dot(oht, lo, preferred_element_type=_F32))
    if exact:
        l2 = (r1 - lo.astype(_F32)).astype(jnp.bfloat16)
        out = out + jnp.dot(oht, l2, preferred_element_type=_F32)
    return out


@functools.cache
def _build_gather2():
    @functools.partial(
        pl.kernel,
        out_type=[jax.ShapeDtypeStruct((_E, _L), _F32),
                  jax.ShapeDtypeStruct((_E, _L), _F32)],
        mesh=plsc.VectorSubcoreMesh(core_axis_name="c", subcore_axis_name="s"),
        scratch_types=[pltpu.VMEM((_RPW,), jnp.int32),
                       pltpu.VMEM((_RPW, _L), _F32),
                       pltpu.SemaphoreType.DMA],
    )
    def k(table, idxa, idxb, outa, outb, idx_v, rows_v, sem):
        wid = lax.axis_index("s") * _NC + lax.axis_index("c")
        base = wid * _RPW
        pltpu.sync_copy(idxa.at[pl.ds(base, _RPW)], idx_v)
        pltpu.async_copy(table.at[idx_v], rows_v, sem).wait()
        pltpu.sync_copy(rows_v, outa.at[pl.ds(base, _RPW)])
        pltpu.sync_copy(idxb.at[pl.ds(base, _RPW)], idx_v)
        pltpu.async_copy(table.at[idx_v], rows_v, sem).wait()
        pltpu.sync_copy(rows_v, outb.at[pl.ds(base, _RPW)])
    return k


@functools.cache
def _build_gather1():
    @functools.partial(
        pl.kernel,
        out_type=jax.ShapeDtypeStruct((_E, _L), _F32),
        mesh=plsc.VectorSubcoreMesh(core_axis_name="c", subcore_axis_name="s"),
        scratch_types=[pltpu.VMEM((_RPW,), jnp.int32),
                       pltpu.VMEM((_RPW, _L), _F32),
                       pltpu.SemaphoreType.DMA],
    )
    def k(table, idxa, outa, idx_v, rows_v, sem):
        wid = lax.axis_index("s") * _NC + lax.axis_index("c")
        base = wid * _RPW
        pltpu.sync_copy(idxa.at[pl.ds(base, _RPW)], idx_v)
        pltpu.async_copy(table.at[idx_v], rows_v, sem).wait()
        pltpu.sync_copy(rows_v, outa.at[pl.ds(base, _RPW)])
    return k


def _sc_gather2(table, idxa, idxb):
    return _build_gather2()(table, idxa, idxb)


def _sc_gather1(table, idxa):
    return _build_gather1()(table, idxa)


def _enc_kernel(xr, ndr, We1r, be1r, We2r, be2r, esr, ebr,
                Wn1r, bn1r, Wn2r, bn2r, nscr, nbir,
                el0r, nl0r, h1r, xher, rser):
    a1 = xr[...] * We1r[...] + be1r[...]
    h1 = jnp.maximum(a1, 0.0)
    a2 = _mmb(h1, We2r[...]) + be2r[...]
    el0, xh, rs = _ln_fwd(a2, esr[...], ebr[...])
    el0r[...] = el0
    h1r[...] = h1
    xher[...] = xh
    rser[...] = rs
    nh = jnp.maximum(_mmb(ndr[...], Wn1r[...]) + bn1r[...], 0.0)
    nl0, _, _ = _ln_fwd(_mmb(nh, Wn2r[...]) + bn2r[...], nscr[...], nbir[...])
    nl0r[...] = nl0


def _proc_edge_kernel(elr, gSr, gRr, Wp1r, bp1r, Wp2r, bp2r, pscr, pbir,
                      elnr, q1r, xhr, rsr):
    el = elr[...]
    m = jnp.concatenate([el, gSr[...], gRr[...]], axis=-1)
    q1 = jnp.maximum(_mmb(m, Wp1r[...]) + bp1r[...], 0.0)
    p2 = _mmb(q1, Wp2r[...]) + bp2r[...]
    d, xh, rs = _ln_fwd(p2, pscr[...], pbir[...])
    elnr[...] = el + d
    q1r[...] = q1
    xhr[...] = xh
    rsr[...] = rs


def _proc_node_kernel(nlr, el1r, rcvr, Wq1r, bq1r, Wq2r, bq2r, qscr, qbir,
                      nlnr, s1r, xhr, rsr):
    nl = nlr[...]
    agg = _seg_sum(rcvr[...], el1r[...], exact=True)
    c = jnp.concatenate([nl, agg], axis=-1)
    s1 = jnp.maximum(_mmb(c, Wq1r[...]) + bq1r[...], 0.0)
    r2 = _mmb(s1, Wq2r[...]) + bq2r[...]
    u, xh, rs = _ln_fwd(r2, qscr[...], qbir[...])
    nlnr[...] = nl + u
    s1r[...] = s1
    xhr[...] = xh
    rsr[...] = rs


def _edge_dec_bwd_kernel(elr, gSr, gRr, Wp1r, bp1r, Wp2r, bp2r, pscr, pbir,
                         Wd1r, bd1r, Wd2r, bd2r, Wd3r,
                         delr, dmsr, dmrr):
    el = elr[...]
    Wp1 = Wp1r[...]
    Wp2 = Wp2r[...]
    psc = pscr[...]
    m = jnp.concatenate([el, gSr[...], gRr[...]], axis=-1)
    q1 = jnp.maximum(_mmb(m, Wp1) + bp1r[...], 0.0)
    p2 = _mmb(q1, Wp2) + bp2r[...]
    d, xh_p, rs_p = _ln_fwd(p2, psc, pbir[...])
    el2 = el + d
    z1 = _mmb(el2, Wd1r[...]) + bd1r[...]
    z2 = _mmb(jnp.maximum(z1, 0.0), Wd2r[...]) + bd2r[...]
    wd3row = Wd3r[...].reshape(1, _L)
    d_z2r = jnp.where(z2 > 0, wd3row, 0.0)
    d_z1 = _mmbT(d_z2r, Wd2r[...]) * (z1 > 0).astype(_F32)
    d_el2 = _mmbT(d_z1, Wd1r[...])
    d_p2 = _ln_bwd(d_el2, xh_p, rs_p, psc)
    d_m = _mmbT(_mmbT(d_p2, Wp2) * (q1 > 0).astype(_F32), Wp1)
    delr[...] = d_el2 + d_m[:, :_L]
    dmsr[...] = d_m[:, _L:2 * _L]
    dmrr[...] = d_m[:, 2 * _L:]


def _node_bwd_kernel(dmsr, dmrr, sndr, rcvr, s1r, xhr, rsr,
                     Wq1r, Wq2r, qscr, daggr):
    d_nl = _seg_sum(sndr[...], dmsr[...]) + _seg_sum(rcvr[...], dmrr[...])
    d_r2 = _ln_bwd(d_nl, xhr[...], rsr[...], qscr[...])
    d_c = _mmbT(_mmbT(d_r2, Wq2r[...]) * (s1r[...] > 0).astype(_F32), Wq1r[...])
    daggr[...] = d_c[:, _L:]


def _final_bwd_kernel(delr, gdaggr, q1r, xhpr, rspr, h1r, xher, rser,
                      Wp1r, Wp2r, pscr, We1r, We2r, esr, dhr):
    d_el_tot = delr[...] + gdaggr[...]
    d_p2 = _ln_bwd(d_el_tot, xhpr[...], rspr[...], pscr[...])
    d_m = _mmbT(_mmbT(d_p2, Wp2r[...]) * (q1r[...] > 0).astype(_F32), Wp1r[...])
    d_el0 = d_el_tot + d_m[:, :_L]
    d_a2 = _ln_bwd(d_el0, xher[...], rser[...], esr[...])
    d_a1 = _mmbT(d_a2, We2r[...]) * (h1r[...] > 0).astype(_F32)
    dhr[...] = jnp.sum(_b16(d_a1).astype(_F32) * _b16(We1r[...]).astype(_F32),
                       axis=-1, keepdims=True)


def _tri_ij(t):
    i = jnp.where(t < 4, 0, jnp.where(t < 7, 1, jnp.where(t < 9, 2, 3)))
    j = jnp.where(t < 4, t, jnp.where(t < 7, t - 3, jnp.where(t < 9, t - 5, 3)))
    return i, j


def _mv1_kernel(wjr, wrr, vr, accr, ur):
    t = pl.program_id(0)
    i, j = _tri_ij(t)

    @pl.when(t == 0)
    def _():
        accr[...] = jnp.zeros_like(accr)
        ur[...] = jnp.zeros_like(ur)

    rg = i * _BLK + lax.broadcasted_iota(jnp.int32, (_BLK, _BLK), 0)
    cg = j * _BLK + lax.broadcasted_iota(jnp.int32, (_BLK, _BLK), 1)
    wjm = jnp.where(rg <= cg, wjr[...], 0.0)
    rgr = j * _BLK + lax.broadcasted_iota(jnp.int32, (_BLK, _BLK), 0)
    cgr = i * _BLK + lax.broadcasted_iota(jnp.int32, (_BLK, _BLK), 1)
    wrm = jnp.where(rgr >= cgr, wrr[...], 0.0)
    vj = vr[pl.ds(j * _BLK, _BLK), :]
    vi = vr[pl.ds(i * _BLK, _BLK), :]
    dnT = (((0,), (0,)), ((), ()))
    accr[pl.ds(i * _BLK, _BLK), :] += _mm(wjm, vj)
    accr[pl.ds(j * _BLK, _BLK), :] += -lax.dot_general(
        wjm, vi, dnT, preferred_element_type=_F32, precision=_HI)
    ur[pl.ds(i * _BLK, _BLK), :] += lax.dot_general(
        wrm, vj, dnT, preferred_element_type=_F32, precision=_HI)


def _mv2_kernel(wrr, wgr, ur, ctlr, accr, xr, outr, rrr):
    i = pl.program_id(0)
    j = pl.program_id(1)

    @pl.when(jnp.logical_and(i == 0, j == 0))
    def _():
        rrr[...] = jnp.zeros_like(rrr)

    rrr[pl.ds(i * _BLK, _BLK), :] += -_mm(wgr[...], ctlr[pl.ds(j * _BLK, _BLK), :])

    @pl.when(j <= i)
    def _():
        rg = i * _BLK + lax.broadcasted_iota(jnp.int32, (_BLK, _BLK), 0)
        cg = j * _BLK + lax.broadcasted_iota(jnp.int32, (_BLK, _BLK), 1)
        wrm = jnp.where(rg >= cg, wrr[...], 0.0)
        rrr[pl.ds(i * _BLK, _BLK), :] += _mm(wrm, ur[pl.ds(j * _BLK, _BLK), :])

    @pl.when(jnp.logical_and(i == _NB - 1, j == _NB - 1))
    def _():
        outr[...] = xr[...] + _DT * (accr[...] - rrr[...])


def kernel(nodes, edges, senders, receivers, control, W_J, W_R, W_g, params):
    x = edges[:, :1]
    snd = senders.astype(jnp.int32)
    rcv = receivers.astype(jnp.int32)

    pe, pn = params["enc_edge"], params["enc_node"]
    pp, pq, pd = params["proc_edge"], params["proc_node"], params["dec_edge"]
    We1, We2 = pe["W"]
    be1, be2 = pe["b"]
    es, eb = pe["ln_scale"], pe["ln_bias"]
    Wn1, Wn2 = pn["W"]
    bn1, bn2 = pn["b"]
    nsc, nbi = pn["ln_scale"], pn["ln_bias"]
    Wp1, Wp2 = pp["W"]
    bp1, bp2 = pp["b"]
    psc, pbi = pp["ln_scale"], pp["ln_bias"]
    Wq1, Wq2 = pq["W"]
    bq1, bq2 = pq["b"]
    qsc, qbi = pq["ln_scale"], pq["ln_bias"]
    Wd1, Wd2, Wd3 = pd["W"]
    bd1, bd2 = pd["b"][0], pd["b"][1]

    f_el = jax.ShapeDtypeStruct((_E, _L), _F32)
    f_nl = jax.ShapeDtypeStruct((_N, _L), _F32)
    f_e1 = jax.ShapeDtypeStruct((_E, 1), _F32)

    el0, nl0, h1, xh_e, rs_e = pl.pallas_call(
        _enc_kernel,
        out_shape=[f_el, f_nl, f_el, f_el, f_e1],
        interpret=_IT,
    )(x, nodes, We1, be1, We2, be2, es, eb, Wn1, bn1, Wn2, bn2, nsc, nbi)

    g0S, g0R = _sc_gather2(nl0, snd, rcv)

    el1, q1_0, xh_p0, rs_p0 = pl.pallas_call(
        _proc_edge_kernel,
        out_shape=[f_el, f_el, f_el, f_e1],
        interpret=_IT,
    )(el0, g0S, g0R, Wp1, bp1, Wp2, bp2, psc, pbi)

    nl1, s1_0, xh_r0, rs_r0 = pl.pallas_call(
        _proc_node_kernel,
        out_shape=[f_nl, f_nl, f_nl, jax.ShapeDtypeStruct((_N, 1), _F32)],
        interpret=_IT,
    )(nl0, el1, rcv, Wq1, bq1, Wq2, bq2, qsc, qbi)

    g1S, g1R = _sc_gather2(nl1, snd, rcv)

    d_el1, d_mS, d_mR = pl.pallas_call(
        _edge_dec_bwd_kernel,
        out_shape=[f_el, f_el, f_el],
        interpret=_IT,
    )(el1, g1S, g1R, Wp1, bp1, Wp2, bp2, psc, pbi,
      Wd1, bd1, Wd2, bd2, Wd3)

    d_agg = pl.pallas_call(
        _node_bwd_kernel,
        out_shape=f_nl,
        interpret=_IT,
    )(d_mS, d_mR, snd, rcv, s1_0, xh_r0, rs_r0, Wq1, Wq2, qsc)

    gDagg = _sc_gather1(d_agg, rcv)

    dH = pl.pallas_call(
        _final_bwd_kernel,
        out_shape=f_e1,
        interpret=_IT,
    )(d_el1, gDagg, q1_0, xh_p0, rs_p0, h1, xh_e, rs_e,
      Wp1, Wp2, psc, We1, We2, es)

    ctl = control[:, None]
    spec_wj = pl.BlockSpec((_BLK, _BLK), lambda t: _tri_ij(t))
    spec_wr1 = pl.BlockSpec((_BLK, _BLK), lambda t: _tri_ij(t)[::-1])
    spec_full1 = pl.BlockSpec((_E, 1), lambda t: (0, 0))
    spec_wg = pl.BlockSpec((_BLK, _BLK), lambda i, j: (i, j))
    spec_wr2 = pl.BlockSpec((_BLK, _BLK), lambda i, j: (i, jnp.minimum(j, i)))
    spec_full = pl.BlockSpec((_E, 1), lambda i, j: (0, 0))

    acc, u = pl.pallas_call(
        _mv1_kernel,
        grid=(_T1,),
        in_specs=[spec_wj, spec_wr1, spec_full1],
        out_specs=[spec_full1, spec_full1],
        out_shape=[f_e1, f_e1],
        interpret=_IT,
    )(W_J, W_R, dH)

    nxt = pl.pallas_call(
        _mv2_kernel,
        grid=(_NB, _NB),
        in_specs=[spec_wr2, spec_wg, spec_full, spec_full, spec_full, spec_full],
        out_specs=spec_full,
        out_shape=f_e1,
        scratch_shapes=[pltpu.VMEM((_E, 1), _F32)],
        interpret=_IT,
    )(W_R, W_g, u, ctl, acc, x)

    return nxt[:, 0]
